# Initial kernel scaffold; baseline (speedup 1.0000x reference)
import functools

import jax
import jax.numpy as jnp
from jax import lax
from jax.experimental import pallas as pl
from jax.experimental.pallas import tpu as pltpu
from jax.experimental.pallas import tpu_sc as plsc

_H, _W, _C = 64, 64, 3
_KH, _KW = 32, 32
_P = 10
_OH = _H + 2 * _P - _KH + 1
_OW = _W + 2 * _P - _KW + 1
_Q = _OH * _OW
_D = _C * _KH * _KW
_M = 10000

_QP = 2816
_BM = 512
_NM = (_M + _BM - 1) // _BM


def _nn_body(q_ref, k_ref, nn_ref, qsq, bv, bi):
    mb = pl.program_id(0)

    @pl.when(mb == 0)
    def _init():
        q = q_ref[...]
        qsq[...] = jnp.sum(q * q, axis=1, keepdims=True)
        bv[...] = jnp.full((_QP, 1), jnp.inf, jnp.float32)
        bi[...] = jnp.zeros((_QP, 1), jnp.int32)

    k = k_ref[...]
    ksq = jnp.sum(k * k, axis=1)
    qk = lax.dot_general(q_ref[...], k, (((1,), (1,)), ((), ())),
                         precision=lax.Precision.HIGHEST,
                         preferred_element_type=jnp.float32)
    d = (qsq[...] - 2.0 * qk) + ksq[None, :]
    ids = lax.broadcasted_iota(jnp.int32, d.shape, 1) + mb * _BM
    d = jnp.where(ids < _M, d, jnp.inf)
    dmin = jnp.min(d, axis=1, keepdims=True)
    imin = jnp.min(jnp.where(d == dmin, ids, _NM * _BM), axis=1, keepdims=True)
    take = dmin < bv[...]
    bi[...] = jnp.where(take, imin, bi[...])
    bv[...] = jnp.where(take, dmin, bv[...])

    @pl.when(mb == _NM - 1)
    def _emit():
        nn_ref[...] = bi[...]


def _nn_search(q, keys):
    return pl.pallas_call(
        _nn_body,
        grid=(_NM,),
        in_specs=[
            pl.BlockSpec((_QP, _D), lambda i: (0, 0)),
            pl.BlockSpec((_BM, _D), lambda i: (i, 0)),
        ],
        out_specs=pl.BlockSpec((_QP, 1), lambda i: (0, 0)),
        out_shape=jax.ShapeDtypeStruct((_QP, 1), jnp.int32),
        scratch_shapes=[
            pltpu.VMEM((_QP, 1), jnp.float32),
            pltpu.VMEM((_QP, 1), jnp.float32),
            pltpu.VMEM((_QP, 1), jnp.int32),
        ],
    )(q, keys)


_NW = 32
_BPW = _QP // _NW
_GC = 8
_NCH = _BPW // _GC


def _gather_body(vals_ref, idx_ref, out_ref, idx_v, rows_a, rows_b, sem_a, sem_b):
    wid = lax.axis_index("s") * 2 + lax.axis_index("c")
    base = wid * _BPW
    pltpu.sync_copy(idx_ref.at[pl.ds(base, _BPW)], idx_v)
    bufs = (rows_a, rows_b)
    sems = (sem_a, sem_b)

    def _start(ch):
        return pltpu.async_copy(
            vals_ref.at[idx_v.at[pl.ds(ch * _GC, _GC)]],
            bufs[ch % 2], sems[ch % 2])

    handles = [None] * _NCH
    handles[0] = _start(0)
    for ch in range(_NCH):
        if ch + 1 < _NCH:
            handles[ch + 1] = _start(ch + 1)
        handles[ch].wait()
        pltpu.sync_copy(bufs[ch % 2], out_ref.at[pl.ds(base + ch * _GC, _GC)])


def _sc_gather(values, idx):
    k = pl.kernel(
        _gather_body,
        out_type=jax.ShapeDtypeStruct((_QP, _D), jnp.float32),
        mesh=plsc.VectorSubcoreMesh(core_axis_name="c", subcore_axis_name="s"),
        scratch_types=[
            pltpu.VMEM((_BPW,), jnp.int32),
            pltpu.VMEM((_GC, _D), jnp.float32),
            pltpu.VMEM((_GC, _D), jnp.float32),
            pltpu.SemaphoreType.DMA,
            pltpu.SemaphoreType.DMA,
        ],
    )
    return k(values, idx)


def _fold_body(pat_ref, out_ref, acc):
    i = pl.program_id(0)

    @pl.when(i == 0)
    def _init():
        acc[...] = jnp.zeros((_C, _H + 2 * _P, _W + 2 * _P), jnp.float32)

    blk = pat_ref[...]
    for dx in range(_KW):
        cur = acc[:, pl.ds(i, _OH), pl.ds(dx, _OW)]
        acc[:, pl.ds(i, _OH), pl.ds(dx, _OW)] = cur + blk[:, 0, dx]

    @pl.when(i == _KH - 1)
    def _fin():
        folded = acc[:, _P:_P + _H, _P:_P + _W]
        out_ref[...] = folded / jnp.max(folded)


def _fold(pat5):
    return pl.pallas_call(
        _fold_body,
        grid=(_KH,),
        in_specs=[pl.BlockSpec((_C, 1, _KW, _OH, _OW),
                               lambda i: (0, i, 0, 0, 0))],
        out_specs=pl.BlockSpec((_C, _H, _W), lambda i: (0, 0, 0)),
        out_shape=jax.ShapeDtypeStruct((_C, _H, _W), jnp.float32),
        scratch_shapes=[pltpu.VMEM((_C, _H + 2 * _P, _W + 2 * _P), jnp.float32)],
    )(pat5)


def kernel(image, mem_keys, mem_values):
    img = jnp.transpose(image, (2, 0, 1))
    padded = jnp.pad(img, ((0, 0), (_P, _P), (_P, _P)))
    hh = jnp.arange(_KH)[:, None] + jnp.arange(_OH)[None, :]
    ww = jnp.arange(_KW)[:, None] + jnp.arange(_OW)[None, :]
    patches = padded[:, hh[:, None, :, None], ww[None, :, None, :]]
    unfolded = patches.reshape(_D, _Q).T
    q = jnp.pad(unfolded, ((0, _QP - _Q), (0, 0)))

    nn = _nn_search(q, mem_keys)[:, 0]
    found = _sc_gather(mem_values, nn)

    pat5 = found[:_Q].T.reshape(_C, _KH, _KW, _OH, _OW)
    out = _fold(pat5)
    return jnp.transpose(out, (1, 2, 0))


# speedup vs baseline: 1.4419x; 1.4419x over previous
import functools

import jax
import jax.numpy as jnp
from jax import lax
from jax.experimental import pallas as pl
from jax.experimental.pallas import tpu as pltpu
from jax.experimental.pallas import tpu_sc as plsc

_H, _W, _C = 64, 64, 3
_KH, _KW = 32, 32
_P = 10
_OH = _H + 2 * _P - _KH + 1
_OW = _W + 2 * _P - _KW + 1
_Q = _OH * _OW
_D = _C * _KH * _KW
_M = 10000

_QP = 2816
_BQ = 704
_NQ = _QP // _BQ
_BM = 256
_NM = (_M + _BM - 1) // _BM


def _nn_body(q_ref, k_ref, qsq_ref, ksq_ref, nn_ref, bv, bi):
    mb = pl.program_id(1)

    @pl.when(mb == 0)
    def _init():
        bv[...] = jnp.full((_BQ, 1), jnp.inf, jnp.float32)
        bi[...] = jnp.zeros((_BQ, 1), jnp.int32)

    qk = lax.dot_general(q_ref[...], k_ref[...], (((1,), (1,)), ((), ())),
                         preferred_element_type=jnp.float32)
    d = (qsq_ref[...] - 2.0 * qk) + ksq_ref[...]
    ids = lax.broadcasted_iota(jnp.int32, d.shape, 1) + mb * _BM
    d = jnp.where(ids < _M, d, jnp.inf)
    dmin = jnp.min(d, axis=1, keepdims=True)
    imin = jnp.min(jnp.where(d == dmin, ids, _NM * _BM), axis=1, keepdims=True)
    take = dmin < bv[...]
    bi[...] = jnp.where(take, imin, bi[...])
    bv[...] = jnp.where(take, dmin, bv[...])

    @pl.when(mb == _NM - 1)
    def _emit():
        nn_ref[...] = bi[...]


def _nn_search(q, keys, qsq, ksq):
    return pl.pallas_call(
        _nn_body,
        grid=(_NQ, _NM),
        in_specs=[
            pl.BlockSpec((_BQ, _D), lambda iq, im: (iq, 0)),
            pl.BlockSpec((_BM, _D), lambda iq, im: (im, 0)),
            pl.BlockSpec((_BQ, 1), lambda iq, im: (iq, 0)),
            pl.BlockSpec((1, _BM), lambda iq, im: (0, im)),
        ],
        out_specs=pl.BlockSpec((_BQ, 1), lambda iq, im: (iq, 0)),
        out_shape=jax.ShapeDtypeStruct((_QP, 1), jnp.int32),
        scratch_shapes=[
            pltpu.VMEM((_BQ, 1), jnp.float32),
            pltpu.VMEM((_BQ, 1), jnp.int32),
        ],
        compiler_params=pltpu.CompilerParams(
            dimension_semantics=("arbitrary", "arbitrary"),
            vmem_limit_bytes=100 * 1024 * 1024),
    )(q, keys, qsq, ksq)


_NW = 32
_BPW = _QP // _NW
_GC = 8
_NCH = _BPW // _GC


def _gather_body(vals_ref, idx_ref, out_ref, idx_v, rows_a, rows_b, sem_a, sem_b):
    wid = lax.axis_index("s") * 2 + lax.axis_index("c")
    base = wid * _BPW
    pltpu.sync_copy(idx_ref.at[pl.ds(base, _BPW)], idx_v)
    bufs = (rows_a, rows_b)
    sems = (sem_a, sem_b)

    def _start(ch):
        return pltpu.async_copy(
            vals_ref.at[idx_v.at[pl.ds(ch * _GC, _GC)]],
            bufs[ch % 2], sems[ch % 2])

    handles = [None] * _NCH
    handles[0] = _start(0)
    for ch in range(_NCH):
        if ch + 1 < _NCH:
            handles[ch + 1] = _start(ch + 1)
        handles[ch].wait()
        pltpu.sync_copy(bufs[ch % 2], out_ref.at[pl.ds(base + ch * _GC, _GC)])


def _sc_gather(values, idx):
    k = pl.kernel(
        _gather_body,
        out_type=jax.ShapeDtypeStruct((_QP, _D), jnp.float32),
        mesh=plsc.VectorSubcoreMesh(core_axis_name="c", subcore_axis_name="s"),
        scratch_types=[
            pltpu.VMEM((_BPW,), jnp.int32),
            pltpu.VMEM((_GC, _D), jnp.float32),
            pltpu.VMEM((_GC, _D), jnp.float32),
            pltpu.SemaphoreType.DMA,
            pltpu.SemaphoreType.DMA,
        ],
    )
    return k(values, idx)


def _fold_body(pat_ref, out_ref, acc):
    i = pl.program_id(0)

    @pl.when(i == 0)
    def _init():
        acc[...] = jnp.zeros((_C, _H + 2 * _P, _W + 2 * _P), jnp.float32)

    blk = pat_ref[...]
    for dx in range(_KW):
        cur = acc[:, pl.ds(i, _OH), pl.ds(dx, _OW)]
        acc[:, pl.ds(i, _OH), pl.ds(dx, _OW)] = cur + blk[:, 0, dx]

    @pl.when(i == _KH - 1)
    def _fin():
        folded = acc[:, _P:_P + _H, _P:_P + _W]
        out_ref[...] = folded / jnp.max(folded)


def _fold(pat5):
    return pl.pallas_call(
        _fold_body,
        grid=(_KH,),
        in_specs=[pl.BlockSpec((_C, 1, _KW, _OH, _OW),
                               lambda i: (0, i, 0, 0, 0))],
        out_specs=pl.BlockSpec((_C, _H, _W), lambda i: (0, 0, 0)),
        out_shape=jax.ShapeDtypeStruct((_C, _H, _W), jnp.float32),
        scratch_shapes=[pltpu.VMEM((_C, _H + 2 * _P, _W + 2 * _P), jnp.float32)],
    )(pat5)


def kernel(image, mem_keys, mem_values):
    img = jnp.transpose(image, (2, 0, 1))
    padded = jnp.pad(img, ((0, 0), (_P, _P), (_P, _P)))
    hh = jnp.arange(_KH)[:, None] + jnp.arange(_OH)[None, :]
    ww = jnp.arange(_KW)[:, None] + jnp.arange(_OW)[None, :]
    patches = padded[:, hh[:, None, :, None], ww[None, :, None, :]]
    unfolded = patches.reshape(_D, _Q).T
    q = jnp.pad(unfolded, ((0, _QP - _Q), (0, 0)))

    qsq = jnp.sum(q * q, axis=1, keepdims=True)
    ksq = jnp.sum(mem_keys * mem_keys, axis=1)[None, :]
    ksq = jnp.pad(ksq, ((0, 0), (0, _NM * _BM - _M)))

    nn = _nn_search(q.astype(jnp.bfloat16),
                    mem_keys.astype(jnp.bfloat16), qsq, ksq)[:, 0]
    found = _sc_gather(mem_values, nn)

    pat5 = found[:_Q].T.reshape(_C, _KH, _KW, _OH, _OW)
    out = _fold(pat5)
    return jnp.transpose(out, (1, 2, 0))


# speedup vs baseline: 46.1495x; 32.0066x over previous
import functools

import jax
import jax.numpy as jnp
from jax import lax
from jax.experimental import pallas as pl
from jax.experimental.pallas import tpu as pltpu
from jax.experimental.pallas import tpu_sc as plsc

_H, _W, _C = 64, 64, 3
_KH, _KW = 32, 32
_P = 10
_OH = _H + 2 * _P - _KH + 1
_OW = _W + 2 * _P - _KW + 1
_Q = _OH * _OW
_D = _C * _KH * _KW
_M = 10000

_QP = 2816
_BQ = 704
_NQ = _QP // _BQ
_BM = 256
_NM = (_M + _BM - 1) // _BM


def _nn_body(q_ref, k_ref, qsq_ref, ksq_ref, nn_ref, bv, bi):
    mb = pl.program_id(1)

    @pl.when(mb == 0)
    def _init():
        bv[...] = jnp.full((_BQ, 1), jnp.inf, jnp.float32)
        bi[...] = jnp.zeros((_BQ, 1), jnp.int32)

    qk = lax.dot_general(q_ref[...], k_ref[...], (((1,), (1,)), ((), ())),
                         preferred_element_type=jnp.float32)
    d = (qsq_ref[...] - 2.0 * qk) + ksq_ref[...]
    ids = lax.broadcasted_iota(jnp.int32, d.shape, 1) + mb * _BM
    d = jnp.where(ids < _M, d, jnp.inf)
    dmin = jnp.min(d, axis=1, keepdims=True)
    imin = jnp.min(jnp.where(d == dmin, ids, _NM * _BM), axis=1, keepdims=True)
    take = dmin < bv[...]
    bi[...] = jnp.where(take, imin, bi[...])
    bv[...] = jnp.where(take, dmin, bv[...])

    @pl.when(mb == _NM - 1)
    def _emit():
        nn_ref[...] = bi[...]


def _nn_search(q, keys, qsq, ksq):
    return pl.pallas_call(
        _nn_body,
        grid=(_NQ, _NM),
        in_specs=[
            pl.BlockSpec((_BQ, _D), lambda iq, im: (iq, 0)),
            pl.BlockSpec((_BM, _D), lambda iq, im: (im, 0)),
            pl.BlockSpec((_BQ, 1), lambda iq, im: (iq, 0)),
            pl.BlockSpec((1, _BM), lambda iq, im: (0, im)),
        ],
        out_specs=pl.BlockSpec((_BQ, 1), lambda iq, im: (iq, 0)),
        out_shape=jax.ShapeDtypeStruct((_QP, 1), jnp.int32),
        scratch_shapes=[
            pltpu.VMEM((_BQ, 1), jnp.float32),
            pltpu.VMEM((_BQ, 1), jnp.int32),
        ],
        compiler_params=pltpu.CompilerParams(
            dimension_semantics=("arbitrary", "arbitrary"),
            vmem_limit_bytes=100 * 1024 * 1024),
    )(q, keys, qsq, ksq)


_NW = 32
_BPW = _QP // _NW
_GC = 8
_NCH = _BPW // _GC


def _gather_body(vals_ref, idx_ref, out_ref, idx_v, rows_a, rows_b, sem_a, sem_b):
    wid = lax.axis_index("s") * 2 + lax.axis_index("c")
    base = wid * _BPW
    pltpu.sync_copy(idx_ref.at[pl.ds(base, _BPW)], idx_v)
    bufs = (rows_a, rows_b)
    sems = (sem_a, sem_b)

    def _start(ch):
        return pltpu.async_copy(
            vals_ref.at[idx_v.at[pl.ds(ch * _GC, _GC)]],
            bufs[ch % 2], sems[ch % 2])

    handles = [None] * _NCH
    handles[0] = _start(0)
    for ch in range(_NCH):
        if ch + 1 < _NCH:
            handles[ch + 1] = _start(ch + 1)
        handles[ch].wait()
        pltpu.sync_copy(bufs[ch % 2], out_ref.at[pl.ds(base + ch * _GC, _GC)])


def _sc_gather(values, idx):
    k = pl.kernel(
        _gather_body,
        out_type=jax.ShapeDtypeStruct((_QP, _D), jnp.float32),
        mesh=plsc.VectorSubcoreMesh(core_axis_name="c", subcore_axis_name="s"),
        scratch_types=[
            pltpu.VMEM((_BPW,), jnp.int32),
            pltpu.VMEM((_GC, _D), jnp.float32),
            pltpu.VMEM((_GC, _D), jnp.float32),
            pltpu.SemaphoreType.DMA,
            pltpu.SemaphoreType.DMA,
        ],
    )
    return k(values, idx)


def _fold_body(pat_ref, out_ref, acc):
    i = pl.program_id(0)

    @pl.when(i == 0)
    def _init():
        acc[...] = jnp.zeros((_C, _H + 2 * _P, _W + 2 * _P), jnp.float32)

    blk = pat_ref[...]
    for dx in range(_KW):
        cur = acc[:, pl.ds(i, _OH), pl.ds(dx, _OW)]
        acc[:, pl.ds(i, _OH), pl.ds(dx, _OW)] = cur + blk[:, 0, dx]

    @pl.when(i == _KH - 1)
    def _fin():
        folded = acc[:, _P:_P + _H, _P:_P + _W]
        out_ref[...] = folded / jnp.max(folded)


def _fold(pat5):
    return pl.pallas_call(
        _fold_body,
        grid=(_KH,),
        in_specs=[pl.BlockSpec((_C, 1, _KW, _OH, _OW),
                               lambda i: (0, i, 0, 0, 0))],
        out_specs=pl.BlockSpec((_C, _H, _W), lambda i: (0, 0, 0)),
        out_shape=jax.ShapeDtypeStruct((_C, _H, _W), jnp.float32),
        scratch_shapes=[pltpu.VMEM((_C, _H + 2 * _P, _W + 2 * _P), jnp.float32)],
    )(pat5)


def kernel(image, mem_keys, mem_values):
    img = jnp.transpose(image, (2, 0, 1))
    padded = jnp.pad(img, ((0, 0), (_P, _P), (_P, _P)))
    sl = jnp.stack([
        jnp.stack([padded[:, dy:dy + _OH, dx:dx + _OW]
                   for dx in range(_KW)], 0)
        for dy in range(_KH)], 0)
    unfolded = jnp.transpose(sl, (3, 4, 2, 0, 1)).reshape(_Q, _D)
    q = jnp.pad(unfolded, ((0, _QP - _Q), (0, 0)))

    qsq = jnp.sum(q * q, axis=1, keepdims=True)
    ksq = jnp.sum(mem_keys * mem_keys, axis=1)[None, :]
    ksq = jnp.pad(ksq, ((0, 0), (0, _NM * _BM - _M)))

    nn = _nn_search(q.astype(jnp.bfloat16),
                    mem_keys.astype(jnp.bfloat16), qsq, ksq)[:, 0]
    found = _sc_gather(mem_values, nn)

    pat5 = found[:_Q].T.reshape(_C, _KH, _KW, _OH, _OW)
    out = _fold(pat5)
    return jnp.transpose(out, (1, 2, 0))


# speedup vs baseline: 49.2790x; 1.0678x over previous
import functools

import jax
import jax.numpy as jnp
from jax import lax
from jax.experimental import pallas as pl
from jax.experimental.pallas import tpu as pltpu
from jax.experimental.pallas import tpu_sc as plsc

_H, _W, _C = 64, 64, 3
_KH, _KW = 32, 32
_P = 10
_OH = _H + 2 * _P - _KH + 1
_OW = _W + 2 * _P - _KW + 1
_Q = _OH * _OW
_D = _C * _KH * _KW
_M = 10000

_QP = 2816
_BQ = 1408
_NQ = _QP // _BQ
_BM = 512
_NM = (_M + _BM - 1) // _BM


def _nn_body(q_ref, k_ref, qsq_ref, ksq_ref, nn_ref, bv, bi):
    mb = pl.program_id(1)

    @pl.when(mb == 0)
    def _init():
        bv[...] = jnp.full((_BQ, 1), jnp.inf, jnp.float32)
        bi[...] = jnp.zeros((_BQ, 1), jnp.int32)

    qk = lax.dot_general(q_ref[...].astype(jnp.bfloat16),
                         k_ref[...].astype(jnp.bfloat16),
                         (((1,), (1,)), ((), ())),
                         preferred_element_type=jnp.float32)
    d = (qsq_ref[...] - 2.0 * qk) + ksq_ref[...]
    ids = lax.broadcasted_iota(jnp.int32, d.shape, 1) + mb * _BM
    d = jnp.where(ids < _M, d, jnp.inf)
    dmin = jnp.min(d, axis=1, keepdims=True)
    imin = jnp.min(jnp.where(d == dmin, ids, _NM * _BM), axis=1, keepdims=True)
    take = dmin < bv[...]
    bi[...] = jnp.where(take, imin, bi[...])
    bv[...] = jnp.where(take, dmin, bv[...])

    @pl.when(mb == _NM - 1)
    def _emit():
        nn_ref[...] = bi[...]


def _nn_search(q, keys, qsq, ksq):
    return pl.pallas_call(
        _nn_body,
        grid=(_NQ, _NM),
        in_specs=[
            pl.BlockSpec((_BQ, _D), lambda iq, im: (iq, 0)),
            pl.BlockSpec((_BM, _D), lambda iq, im: (im, 0)),
            pl.BlockSpec((_BQ, 1), lambda iq, im: (iq, 0)),
            pl.BlockSpec((1, _BM), lambda iq, im: (0, im)),
        ],
        out_specs=pl.BlockSpec((_BQ, 1), lambda iq, im: (iq, 0)),
        out_shape=jax.ShapeDtypeStruct((_QP, 1), jnp.int32),
        scratch_shapes=[
            pltpu.VMEM((_BQ, 1), jnp.float32),
            pltpu.VMEM((_BQ, 1), jnp.int32),
        ],
        compiler_params=pltpu.CompilerParams(
            dimension_semantics=("arbitrary", "arbitrary"),
            vmem_limit_bytes=100 * 1024 * 1024),
    )(q, keys, qsq, ksq)


_NW = 32
_BPW = _QP // _NW
_GC = 8
_NCH = _BPW // _GC


def _gather_body(vals_ref, idx_ref, out_ref, idx_v, rows_a, rows_b, sem_a, sem_b):
    wid = lax.axis_index("s") * 2 + lax.axis_index("c")
    base = wid * _BPW
    pltpu.sync_copy(idx_ref.at[pl.ds(base, _BPW)], idx_v)
    bufs = (rows_a, rows_b)
    sems = (sem_a, sem_b)

    def _start(ch):
        return pltpu.async_copy(
            vals_ref.at[idx_v.at[pl.ds(ch * _GC, _GC)]],
            bufs[ch % 2], sems[ch % 2])

    handles = [None] * _NCH
    handles[0] = _start(0)
    for ch in range(_NCH):
        if ch + 1 < _NCH:
            handles[ch + 1] = _start(ch + 1)
        handles[ch].wait()
        pltpu.sync_copy(bufs[ch % 2], out_ref.at[pl.ds(base + ch * _GC, _GC)])


def _sc_gather(values, idx):
    k = pl.kernel(
        _gather_body,
        out_type=jax.ShapeDtypeStruct((_QP, _D), jnp.float32),
        mesh=plsc.VectorSubcoreMesh(core_axis_name="c", subcore_axis_name="s"),
        scratch_types=[
            pltpu.VMEM((_BPW,), jnp.int32),
            pltpu.VMEM((_GC, _D), jnp.float32),
            pltpu.VMEM((_GC, _D), jnp.float32),
            pltpu.SemaphoreType.DMA,
            pltpu.SemaphoreType.DMA,
        ],
    )
    return k(values, idx)


def _fold_body(pat_ref, out_ref, acc):
    i = pl.program_id(0)

    @pl.when(i == 0)
    def _init():
        acc[...] = jnp.zeros((_C, _H + 2 * _P, _W + 2 * _P), jnp.float32)

    blk = pat_ref[...]
    for dx in range(_KW):
        cur = acc[:, pl.ds(i, _OH), pl.ds(dx, _OW)]
        acc[:, pl.ds(i, _OH), pl.ds(dx, _OW)] = cur + blk[:, 0, dx]

    @pl.when(i == _KH - 1)
    def _fin():
        folded = acc[:, _P:_P + _H, _P:_P + _W]
        out_ref[...] = folded / jnp.max(folded)


def _fold(pat5):
    return pl.pallas_call(
        _fold_body,
        grid=(_KH,),
        in_specs=[pl.BlockSpec((_C, 1, _KW, _OH, _OW),
                               lambda i: (0, i, 0, 0, 0))],
        out_specs=pl.BlockSpec((_C, _H, _W), lambda i: (0, 0, 0)),
        out_shape=jax.ShapeDtypeStruct((_C, _H, _W), jnp.float32),
        scratch_shapes=[pltpu.VMEM((_C, _H + 2 * _P, _W + 2 * _P), jnp.float32)],
    )(pat5)


def kernel(image, mem_keys, mem_values):
    img = jnp.transpose(image, (2, 0, 1))
    padded = jnp.pad(img, ((0, 0), (_P, _P), (_P, _P)))
    sl = jnp.stack([
        jnp.stack([padded[:, dy:dy + _OH, dx:dx + _OW]
                   for dx in range(_KW)], 0)
        for dy in range(_KH)], 0)
    unfolded = jnp.transpose(sl, (3, 4, 2, 0, 1)).reshape(_Q, _D)
    q = jnp.pad(unfolded, ((0, _QP - _Q), (0, 0)))

    qsq = jnp.sum(q * q, axis=1, keepdims=True)
    ksq = jnp.sum(mem_keys * mem_keys, axis=1)[None, :]
    ksq = jnp.pad(ksq, ((0, 0), (0, _NM * _BM - _M)))

    nn = _nn_search(q, mem_keys, qsq, ksq)[:, 0]
    found = _sc_gather(mem_values, nn)

    pat5 = found[:_Q].T.reshape(_C, _KH, _KW, _OH, _OW)
    out = _fold(pat5)
    return jnp.transpose(out, (1, 2, 0))


# speedup vs baseline: 53.5718x; 1.0871x over previous
import functools

import jax
import jax.numpy as jnp
from jax import lax
from jax.experimental import pallas as pl
from jax.experimental.pallas import tpu as pltpu
from jax.experimental.pallas import tpu_sc as plsc

_H, _W, _C = 64, 64, 3
_KH, _KW = 32, 32
_P = 10
_OH = _H + 2 * _P - _KH + 1
_OW = _W + 2 * _P - _KW + 1
_Q = _OH * _OW
_D = _C * _KH * _KW
_M = 10000

_QP = 2816
_BQ = 1408
_NQ = _QP // _BQ
_BM = 512
_NM = (_M + _BM - 1) // _BM


def _nn_body(q_ref, k_ref, qsq_ref, ksq_ref, nn_ref, bv, bi):
    mb = pl.program_id(1)

    @pl.when(mb == 0)
    def _init():
        bv[...] = jnp.full((_BQ, 1), jnp.inf, jnp.float32)
        bi[...] = jnp.zeros((_BQ, 1), jnp.int32)

    qk = lax.dot_general(q_ref[...].astype(jnp.bfloat16),
                         k_ref[...].astype(jnp.bfloat16),
                         (((1,), (1,)), ((), ())),
                         preferred_element_type=jnp.float32)
    d = (qsq_ref[...] - 2.0 * qk) + ksq_ref[...]
    ids = lax.broadcasted_iota(jnp.int32, d.shape, 1) + mb * _BM
    d = jnp.where(ids < _M, d, jnp.inf)
    dmin = jnp.min(d, axis=1, keepdims=True)
    imin = jnp.min(jnp.where(d == dmin, ids, _NM * _BM), axis=1, keepdims=True)
    take = dmin < bv[...]
    bi[...] = jnp.where(take, imin, bi[...])
    bv[...] = jnp.where(take, dmin, bv[...])

    @pl.when(mb == _NM - 1)
    def _emit():
        nn_ref[...] = bi[...]


def _nn_search(q, keys, qsq, ksq):
    return pl.pallas_call(
        _nn_body,
        grid=(_NQ, _NM),
        in_specs=[
            pl.BlockSpec((_BQ, _D), lambda iq, im: (iq, 0)),
            pl.BlockSpec((_BM, _D), lambda iq, im: (im, 0)),
            pl.BlockSpec((_BQ, 1), lambda iq, im: (iq, 0)),
            pl.BlockSpec((1, _BM), lambda iq, im: (0, im)),
        ],
        out_specs=pl.BlockSpec((_BQ, 1), lambda iq, im: (iq, 0)),
        out_shape=jax.ShapeDtypeStruct((_QP, 1), jnp.int32),
        scratch_shapes=[
            pltpu.VMEM((_BQ, 1), jnp.float32),
            pltpu.VMEM((_BQ, 1), jnp.int32),
        ],
        compiler_params=pltpu.CompilerParams(
            dimension_semantics=("arbitrary", "arbitrary"),
            vmem_limit_bytes=100 * 1024 * 1024),
    )(q, keys, qsq, ksq)


_NW = 32
_BPW = _QP // _NW
_GC = 8
_NCH = _BPW // _GC


def _gather_body(vals_ref, idx_ref, out_ref, idx_v, rows_a, rows_b, sem_a, sem_b):
    wid = lax.axis_index("s") * 2 + lax.axis_index("c")
    base = wid * _BPW
    pltpu.sync_copy(idx_ref.at[pl.ds(base, _BPW)], idx_v)
    bufs = (rows_a, rows_b)
    sems = (sem_a, sem_b)

    def _start(ch):
        return pltpu.async_copy(
            vals_ref.at[idx_v.at[pl.ds(ch * _GC, _GC)]],
            bufs[ch % 2], sems[ch % 2])

    handles = [None] * _NCH
    handles[0] = _start(0)
    for ch in range(_NCH):
        if ch + 1 < _NCH:
            handles[ch + 1] = _start(ch + 1)
        handles[ch].wait()
        pltpu.sync_copy(bufs[ch % 2], out_ref.at[pl.ds(base + ch * _GC, _GC)])


def _sc_gather(values, idx):
    k = pl.kernel(
        _gather_body,
        out_type=jax.ShapeDtypeStruct((_QP, _D), jnp.float32),
        mesh=plsc.VectorSubcoreMesh(core_axis_name="c", subcore_axis_name="s"),
        scratch_types=[
            pltpu.VMEM((_BPW,), jnp.int32),
            pltpu.VMEM((_GC, _D), jnp.float32),
            pltpu.VMEM((_GC, _D), jnp.float32),
            pltpu.SemaphoreType.DMA,
            pltpu.SemaphoreType.DMA,
        ],
    )
    return k(values, idx)


_RPT = _QP // _NW
_GCH = 8
_NG = _D // 16
_AR, _AC = _C * 84, 96


def _fold_sc_body(vals_ref, idx_ref, part_ref, idx_v, rows_v, acc, sem):
    wid = lax.axis_index("s") * 2 + lax.axis_index("c")
    base = wid * _RPT
    pltpu.sync_copy(idx_ref.at[pl.ds(base, _RPT)], idx_v)

    def _zero(r, _):
        for g in range(_AC // 16):
            acc[r, pl.ds(g * 16, 16)] = jnp.zeros((16,), jnp.float32)
        return 0

    lax.fori_loop(0, _AR, _zero, 0)

    nrows = _RPT - (_QP - _Q) * (wid // (_NW - 1))

    def _row(i, _):
        ch = i // _GCH

        @pl.when(i == ch * _GCH)
        def _fetch():
            pltpu.async_copy(
                vals_ref.at[idx_v.at[pl.ds(pl.multiple_of(ch * _GCH, 8), _GCH)]],
                rows_v, sem).wait()

        r = i - ch * _GCH
        qi = base + i
        oy = lax.shift_right_logical(qi * 19785, 20)
        ox = qi - oy * _OW
        for c in range(_C):
            for dy in range(_KH):
                row = c * 84 + dy + oy
                db = c * _KH * _KW + dy * _KW
                for h in range(2):
                    v = rows_v[r, pl.ds(db + h * 16, 16)]
                    cur = acc[row, pl.ds(ox + h * 16, 16)]
                    acc[row, pl.ds(ox + h * 16, 16)] = cur + v
        return 0

    lax.fori_loop(0, nrows, _row, 0)
    pltpu.sync_copy(acc, part_ref.at[wid])


def _fold_sc(values, idx):
    k = pl.kernel(
        _fold_sc_body,
        out_type=jax.ShapeDtypeStruct((_NW, _AR, _AC), jnp.float32),
        mesh=plsc.VectorSubcoreMesh(core_axis_name="c", subcore_axis_name="s"),
        scratch_types=[
            pltpu.VMEM((_RPT,), jnp.int32),
            pltpu.VMEM((_GCH, _D), jnp.float32),
            pltpu.VMEM((_AR, _AC), jnp.float32),
            pltpu.SemaphoreType.DMA,
        ],
    )
    return k(values, idx)


def _merge_body(p_ref, out_ref):
    s = jnp.sum(p_ref[...], axis=0)
    folded = jnp.stack(
        [s[c * 84 + _P:c * 84 + _P + _H, _P:_P + _W] for c in range(_C)], 0)
    out_ref[...] = folded / jnp.max(folded)


def _merge(partials):
    return pl.pallas_call(
        _merge_body,
        out_shape=jax.ShapeDtypeStruct((_C, _H, _W), jnp.float32),
    )(partials)


def _fold_body(pat_ref, out_ref, acc):
    i = pl.program_id(0)

    @pl.when(i == 0)
    def _init():
        acc[...] = jnp.zeros((_C, _H + 2 * _P, _W + 2 * _P), jnp.float32)

    blk = pat_ref[...]
    for dx in range(_KW):
        cur = acc[:, pl.ds(i, _OH), pl.ds(dx, _OW)]
        acc[:, pl.ds(i, _OH), pl.ds(dx, _OW)] = cur + blk[:, 0, dx]

    @pl.when(i == _KH - 1)
    def _fin():
        folded = acc[:, _P:_P + _H, _P:_P + _W]
        out_ref[...] = folded / jnp.max(folded)


def _fold(pat5):
    return pl.pallas_call(
        _fold_body,
        grid=(_KH,),
        in_specs=[pl.BlockSpec((_C, 1, _KW, _OH, _OW),
                               lambda i: (0, i, 0, 0, 0))],
        out_specs=pl.BlockSpec((_C, _H, _W), lambda i: (0, 0, 0)),
        out_shape=jax.ShapeDtypeStruct((_C, _H, _W), jnp.float32),
        scratch_shapes=[pltpu.VMEM((_C, _H + 2 * _P, _W + 2 * _P), jnp.float32)],
    )(pat5)


def kernel(image, mem_keys, mem_values):
    img = jnp.transpose(image, (2, 0, 1))
    padded = jnp.pad(img, ((0, 0), (_P, _P), (_P, _P)))
    sl = jnp.stack([
        jnp.stack([padded[:, dy:dy + _OH, dx:dx + _OW]
                   for dx in range(_KW)], 0)
        for dy in range(_KH)], 0)
    unfolded = jnp.transpose(sl, (3, 4, 2, 0, 1)).reshape(_Q, _D)
    q = jnp.pad(unfolded, ((0, _QP - _Q), (0, 0)))

    qsq = jnp.sum(q * q, axis=1, keepdims=True)
    ksq = jnp.sum(mem_keys * mem_keys, axis=1)[None, :]
    ksq = jnp.pad(ksq, ((0, 0), (0, _NM * _BM - _M)))

    nn = _nn_search(q, mem_keys, qsq, ksq)[:, 0]

    partials = _fold_sc(mem_values, nn)
    out = _merge(partials)
    return jnp.transpose(out, (1, 2, 0))


# speedup vs baseline: 99.0478x; 1.8489x over previous
import functools

import jax
import jax.numpy as jnp
from jax import lax
from jax.experimental import pallas as pl
from jax.experimental.pallas import tpu as pltpu
from jax.experimental.pallas import tpu_sc as plsc

_H, _W, _C = 64, 64, 3
_KH, _KW = 32, 32
_P = 10
_OH = _H + 2 * _P - _KH + 1
_OW = _W + 2 * _P - _KW + 1
_Q = _OH * _OW
_D = _C * _KH * _KW
_M = 10000

_QP = 2816
_BQ = 1408
_NQ = _QP // _BQ
_BM = 512
_NM = (_M + _BM - 1) // _BM


def _nn_body(q_ref, k_ref, qsq_ref, ksq_ref, nn_ref, bv, bi):
    mb = pl.program_id(1)

    @pl.when(mb == 0)
    def _init():
        bv[...] = jnp.full((_BQ, 1), jnp.inf, jnp.float32)
        bi[...] = jnp.zeros((_BQ, 1), jnp.int32)

    qk = lax.dot_general(q_ref[...].astype(jnp.bfloat16),
                         k_ref[...].astype(jnp.bfloat16),
                         (((1,), (1,)), ((), ())),
                         preferred_element_type=jnp.float32)
    d = (qsq_ref[...] - 2.0 * qk) + ksq_ref[...]
    ids = lax.broadcasted_iota(jnp.int32, d.shape, 1) + mb * _BM
    d = jnp.where(ids < _M, d, jnp.inf)
    dmin = jnp.min(d, axis=1, keepdims=True)
    imin = jnp.min(jnp.where(d == dmin, ids, _NM * _BM), axis=1, keepdims=True)
    take = dmin < bv[...]
    bi[...] = jnp.where(take, imin, bi[...])
    bv[...] = jnp.where(take, dmin, bv[...])

    @pl.when(mb == _NM - 1)
    def _emit():
        nn_ref[...] = bi[...]


def _nn_search(q, keys, qsq, ksq):
    return pl.pallas_call(
        _nn_body,
        grid=(_NQ, _NM),
        in_specs=[
            pl.BlockSpec((_BQ, _D), lambda iq, im: (iq, 0)),
            pl.BlockSpec((_BM, _D), lambda iq, im: (im, 0)),
            pl.BlockSpec((_BQ, 1), lambda iq, im: (iq, 0)),
            pl.BlockSpec((1, _BM), lambda iq, im: (0, im)),
        ],
        out_specs=pl.BlockSpec((_BQ, 1), lambda iq, im: (iq, 0)),
        out_shape=jax.ShapeDtypeStruct((_QP, 1), jnp.int32),
        scratch_shapes=[
            pltpu.VMEM((_BQ, 1), jnp.float32),
            pltpu.VMEM((_BQ, 1), jnp.int32),
        ],
        compiler_params=pltpu.CompilerParams(
            dimension_semantics=("arbitrary", "arbitrary"),
            vmem_limit_bytes=100 * 1024 * 1024),
    )(q, keys, qsq, ksq)


_NW = 32
_BPW = _QP // _NW
_GC = 8
_NCH = _BPW // _GC


def _gather_body(vals_ref, idx_ref, out_ref, idx_v, rows_a, rows_b, sem_a, sem_b):
    wid = lax.axis_index("s") * 2 + lax.axis_index("c")
    base = wid * _BPW
    pltpu.sync_copy(idx_ref.at[pl.ds(base, _BPW)], idx_v)
    bufs = (rows_a, rows_b)
    sems = (sem_a, sem_b)

    def _start(ch):
        return pltpu.async_copy(
            vals_ref.at[idx_v.at[pl.ds(ch * _GC, _GC)]],
            bufs[ch % 2], sems[ch % 2])

    handles = [None] * _NCH
    handles[0] = _start(0)
    for ch in range(_NCH):
        if ch + 1 < _NCH:
            handles[ch + 1] = _start(ch + 1)
        handles[ch].wait()
        pltpu.sync_copy(bufs[ch % 2], out_ref.at[pl.ds(base + ch * _GC, _GC)])


def _sc_gather(values, idx):
    k = pl.kernel(
        _gather_body,
        out_type=jax.ShapeDtypeStruct((_QP, _D), jnp.float32),
        mesh=plsc.VectorSubcoreMesh(core_axis_name="c", subcore_axis_name="s"),
        scratch_types=[
            pltpu.VMEM((_BPW,), jnp.int32),
            pltpu.VMEM((_GC, _D), jnp.float32),
            pltpu.VMEM((_GC, _D), jnp.float32),
            pltpu.SemaphoreType.DMA,
            pltpu.SemaphoreType.DMA,
        ],
    )
    return k(values, idx)


def _unfold_sc_body(img_ref, q_ref, img_v, row_v, sem):
    wid = lax.axis_index("s") * 2 + lax.axis_index("c")
    base = wid * _RPT
    pltpu.sync_copy(img_ref, img_v)

    nrows = _RPT - (_QP - _Q) * (wid // (_NW - 1))

    def _row(i, _):
        r = i - (i // _GCH) * _GCH
        qi = base + i
        oy = lax.shift_right_logical(qi * 19785, 20)
        ox = qi - oy * _OW
        off0 = oy * 84 + ox
        for c in range(_C):
            for dy in range(_KH):
                src = off0 + c * 7056 + dy * 84
                db = c * _KH * _KW + dy * _KW
                for h in range(2):
                    row_v[r, pl.ds(db + h * 16, 16)] = img_v[pl.ds(src + h * 16, 16)]

        @pl.when(i - (i // _GCH) * _GCH == _GCH - 1)
        def _flush():
            pltpu.sync_copy(
                row_v, q_ref.at[pl.ds(pl.multiple_of((i // _GCH) * _GCH + base, 8),
                                      _GCH)])
        return 0

    lax.fori_loop(0, nrows, _row, 0)

    @pl.when(wid == _NW - 1)
    def _tail():
        pltpu.sync_copy(row_v, q_ref.at[pl.ds(pl.multiple_of(base + 80, 8), _GCH)])


def _unfold_sc(padimg):
    k = pl.kernel(
        _unfold_sc_body,
        out_type=jax.ShapeDtypeStruct((_QP, _D), jnp.float32),
        mesh=plsc.VectorSubcoreMesh(core_axis_name="c", subcore_axis_name="s"),
        scratch_types=[
            pltpu.VMEM((_C * 84 * 84,), jnp.float32),
            pltpu.VMEM((_GCH, _D), jnp.float32),
            pltpu.SemaphoreType.DMA,
        ],
    )
    return k(padimg)


_RPT = _QP // _NW
_GCH = 8
_NG = _D // 16
_AR, _AC = _C * 84, 96


def _fold_sc_body(vals_ref, idx_ref, part_ref, idx_v, rows_v, acc, sem):
    wid = lax.axis_index("s") * 2 + lax.axis_index("c")
    base = wid * _RPT
    pltpu.sync_copy(idx_ref.at[pl.ds(base, _RPT)], idx_v)

    def _zero(r, _):
        for g in range(_AC // 16):
            acc[r, pl.ds(g * 16, 16)] = jnp.zeros((16,), jnp.float32)
        return 0

    lax.fori_loop(0, _AR, _zero, 0)

    nrows = _RPT - (_QP - _Q) * (wid // (_NW - 1))

    def _row(i, _):
        ch = i // _GCH

        @pl.when(i == ch * _GCH)
        def _fetch():
            pltpu.async_copy(
                vals_ref.at[idx_v.at[pl.ds(pl.multiple_of(ch * _GCH, 8), _GCH)]],
                rows_v, sem).wait()

        r = i - ch * _GCH
        qi = base + i
        oy = lax.shift_right_logical(qi * 19785, 20)
        ox = qi - oy * _OW
        for c in range(_C):
            for dy in range(_KH):
                row = c * 84 + dy + oy
                db = c * _KH * _KW + dy * _KW
                for h in range(2):
                    v = rows_v[r, pl.ds(db + h * 16, 16)]
                    cur = acc[row, pl.ds(ox + h * 16, 16)]
                    acc[row, pl.ds(ox + h * 16, 16)] = cur + v
        return 0

    lax.fori_loop(0, nrows, _row, 0)
    pltpu.sync_copy(acc, part_ref.at[wid])


def _fold_sc(values, idx):
    k = pl.kernel(
        _fold_sc_body,
        out_type=jax.ShapeDtypeStruct((_NW, _AR, _AC), jnp.float32),
        mesh=plsc.VectorSubcoreMesh(core_axis_name="c", subcore_axis_name="s"),
        scratch_types=[
            pltpu.VMEM((_RPT,), jnp.int32),
            pltpu.VMEM((_GCH, _D), jnp.float32),
            pltpu.VMEM((_AR, _AC), jnp.float32),
            pltpu.SemaphoreType.DMA,
        ],
    )
    return k(values, idx)


def _merge_body(p_ref, out_ref):
    s = jnp.sum(p_ref[...], axis=0)
    folded = jnp.stack(
        [s[c * 84 + _P:c * 84 + _P + _H, _P:_P + _W] for c in range(_C)], 0)
    out_ref[...] = folded / jnp.max(folded)


def _merge(partials):
    return pl.pallas_call(
        _merge_body,
        out_shape=jax.ShapeDtypeStruct((_C, _H, _W), jnp.float32),
    )(partials)


def _fold_body(pat_ref, out_ref, acc):
    i = pl.program_id(0)

    @pl.when(i == 0)
    def _init():
        acc[...] = jnp.zeros((_C, _H + 2 * _P, _W + 2 * _P), jnp.float32)

    blk = pat_ref[...]
    for dx in range(_KW):
        cur = acc[:, pl.ds(i, _OH), pl.ds(dx, _OW)]
        acc[:, pl.ds(i, _OH), pl.ds(dx, _OW)] = cur + blk[:, 0, dx]

    @pl.when(i == _KH - 1)
    def _fin():
        folded = acc[:, _P:_P + _H, _P:_P + _W]
        out_ref[...] = folded / jnp.max(folded)


def _fold(pat5):
    return pl.pallas_call(
        _fold_body,
        grid=(_KH,),
        in_specs=[pl.BlockSpec((_C, 1, _KW, _OH, _OW),
                               lambda i: (0, i, 0, 0, 0))],
        out_specs=pl.BlockSpec((_C, _H, _W), lambda i: (0, 0, 0)),
        out_shape=jax.ShapeDtypeStruct((_C, _H, _W), jnp.float32),
        scratch_shapes=[pltpu.VMEM((_C, _H + 2 * _P, _W + 2 * _P), jnp.float32)],
    )(pat5)


def kernel(image, mem_keys, mem_values):
    img = jnp.transpose(image, (2, 0, 1))
    padded = jnp.pad(img, ((0, 0), (_P, _P), (_P, _P)))
    q = _unfold_sc(padded.reshape(_C * 84 * 84))

    qsq = jnp.sum(q * q, axis=1, keepdims=True)
    ksq = jnp.sum(mem_keys * mem_keys, axis=1)[None, :]
    ksq = jnp.pad(ksq, ((0, 0), (0, _NM * _BM - _M)))

    nn = _nn_search(q, mem_keys, qsq, ksq)[:, 0]

    partials = _fold_sc(mem_values, nn)
    out = _merge(partials)
    return jnp.transpose(out, (1, 2, 0))


# speedup vs baseline: 100.6831x; 1.0165x over previous
import functools

import jax
import jax.numpy as jnp
from jax import lax
from jax.experimental import pallas as pl
from jax.experimental.pallas import tpu as pltpu
from jax.experimental.pallas import tpu_sc as plsc

_H, _W, _C = 64, 64, 3
_KH, _KW = 32, 32
_P = 10
_OH = _H + 2 * _P - _KH + 1
_OW = _W + 2 * _P - _KW + 1
_Q = _OH * _OW
_D = _C * _KH * _KW
_M = 10000

_QP = 2816
_BQ = 2816
_NQ = _QP // _BQ
_BM = 512
_NM = (_M + _BM - 1) // _BM


def _nn_body(q_ref, k_ref, qsq_ref, ksq_ref, nn_ref, bv, bi):
    mb = pl.program_id(1)

    @pl.when(mb == 0)
    def _init():
        bv[...] = jnp.full((_BQ, 1), jnp.inf, jnp.float32)
        bi[...] = jnp.zeros((_BQ, 1), jnp.int32)

    qk = lax.dot_general(q_ref[...],
                         k_ref[...].astype(jnp.bfloat16),
                         (((1,), (1,)), ((), ())),
                         preferred_element_type=jnp.float32)
    d = (qsq_ref[...] - 2.0 * qk) + ksq_ref[...]
    ids = lax.broadcasted_iota(jnp.int32, d.shape, 1) + mb * _BM
    d = jnp.where(ids < _M, d, jnp.inf)
    dmin = jnp.min(d, axis=1, keepdims=True)
    imin = jnp.min(jnp.where(d == dmin, ids, _NM * _BM), axis=1, keepdims=True)
    take = dmin < bv[...]
    bi[...] = jnp.where(take, imin, bi[...])
    bv[...] = jnp.where(take, dmin, bv[...])

    @pl.when(mb == _NM - 1)
    def _emit():
        nn_ref[...] = bi[...]


def _nn_search(q, keys, qsq, ksq):
    return pl.pallas_call(
        _nn_body,
        grid=(_NQ, _NM),
        in_specs=[
            pl.BlockSpec((_BQ, _D), lambda iq, im: (iq, 0)),
            pl.BlockSpec((_BM, _D), lambda iq, im: (im, 0)),
            pl.BlockSpec((_BQ, 1), lambda iq, im: (iq, 0)),
            pl.BlockSpec((1, _BM), lambda iq, im: (0, im)),
        ],
        out_specs=pl.BlockSpec((_BQ, 1), lambda iq, im: (iq, 0)),
        out_shape=jax.ShapeDtypeStruct((_QP, 1), jnp.int32),
        scratch_shapes=[
            pltpu.VMEM((_BQ, 1), jnp.float32),
            pltpu.VMEM((_BQ, 1), jnp.int32),
        ],
        compiler_params=pltpu.CompilerParams(
            dimension_semantics=("arbitrary", "arbitrary"),
            vmem_limit_bytes=100 * 1024 * 1024),
    )(q, keys, qsq, ksq)


_NW = 32
_BPW = _QP // _NW
_GC = 8
_NCH = _BPW // _GC


def _gather_body(vals_ref, idx_ref, out_ref, idx_v, rows_a, rows_b, sem_a, sem_b):
    wid = lax.axis_index("s") * 2 + lax.axis_index("c")
    base = wid * _BPW
    pltpu.sync_copy(idx_ref.at[pl.ds(base, _BPW)], idx_v)
    bufs = (rows_a, rows_b)
    sems = (sem_a, sem_b)

    def _start(ch):
        return pltpu.async_copy(
            vals_ref.at[idx_v.at[pl.ds(ch * _GC, _GC)]],
            bufs[ch % 2], sems[ch % 2])

    handles = [None] * _NCH
    handles[0] = _start(0)
    for ch in range(_NCH):
        if ch + 1 < _NCH:
            handles[ch + 1] = _start(ch + 1)
        handles[ch].wait()
        pltpu.sync_copy(bufs[ch % 2], out_ref.at[pl.ds(base + ch * _GC, _GC)])


def _sc_gather(values, idx):
    k = pl.kernel(
        _gather_body,
        out_type=jax.ShapeDtypeStruct((_QP, _D), jnp.float32),
        mesh=plsc.VectorSubcoreMesh(core_axis_name="c", subcore_axis_name="s"),
        scratch_types=[
            pltpu.VMEM((_BPW,), jnp.int32),
            pltpu.VMEM((_GC, _D), jnp.float32),
            pltpu.VMEM((_GC, _D), jnp.float32),
            pltpu.SemaphoreType.DMA,
            pltpu.SemaphoreType.DMA,
        ],
    )
    return k(values, idx)


def _unfold_sc_body(img_ref, q_ref, img_v, row_v, sem):
    wid = lax.axis_index("s") * 2 + lax.axis_index("c")
    base = wid * _RPT
    pltpu.sync_copy(img_ref, img_v)

    nrows = _RPT - (_QP - _Q) * (wid // (_NW - 1))

    def _row(i, _):
        r = i - (i // _GCH) * _GCH
        qi = base + i
        oy = lax.shift_right_logical(qi * 19785, 20)
        ox = qi - oy * _OW
        off0 = oy * 84 + ox
        for c in range(_C):
            for dy in range(_KH):
                src = off0 + c * 7056 + dy * 84
                db = c * _KH * _KW + dy * _KW
                for h in range(2):
                    row_v[r, pl.ds(db + h * 16, 16)] = img_v[pl.ds(src + h * 16, 16)]

        @pl.when(i - (i // _GCH) * _GCH == _GCH - 1)
        def _flush():
            pltpu.sync_copy(
                row_v, q_ref.at[pl.ds(pl.multiple_of((i // _GCH) * _GCH + base, 8),
                                      _GCH)])
        return 0

    lax.fori_loop(0, nrows, _row, 0)

    @pl.when(wid == _NW - 1)
    def _tail():
        pltpu.sync_copy(row_v, q_ref.at[pl.ds(pl.multiple_of(base + 80, 8), _GCH)])


def _unfold_sc(padimg):
    k = pl.kernel(
        _unfold_sc_body,
        out_type=jax.ShapeDtypeStruct((_QP, _D), jnp.float32),
        mesh=plsc.VectorSubcoreMesh(core_axis_name="c", subcore_axis_name="s"),
        scratch_types=[
            pltpu.VMEM((_C * 84 * 84,), jnp.float32),
            pltpu.VMEM((_GCH, _D), jnp.float32),
            pltpu.SemaphoreType.DMA,
        ],
    )
    return k(padimg)


_RPT = _QP // _NW
_GCH = 8
_NG = _D // 16
_AR, _AC = _C * 84, 96


def _fold_sc_body(vals_ref, idx_ref, part_ref, idx_v, rows_v, acc, sem):
    wid = lax.axis_index("s") * 2 + lax.axis_index("c")
    base = wid * _RPT
    pltpu.sync_copy(idx_ref.at[pl.ds(base, _RPT)], idx_v)

    def _zero(r, _):
        for g in range(_AC // 16):
            acc[r, pl.ds(g * 16, 16)] = jnp.zeros((16,), jnp.float32)
        return 0

    lax.fori_loop(0, _AR, _zero, 0)

    nrows = _RPT - (_QP - _Q) * (wid // (_NW - 1))

    def _row(i, _):
        ch = i // _GCH

        @pl.when(i == ch * _GCH)
        def _fetch():
            pltpu.async_copy(
                vals_ref.at[idx_v.at[pl.ds(pl.multiple_of(ch * _GCH, 8), _GCH)]],
                rows_v, sem).wait()

        r = i - ch * _GCH
        qi = base + i
        oy = lax.shift_right_logical(qi * 19785, 20)
        ox = qi - oy * _OW
        for c in range(_C):
            for dy in range(_KH):
                row = c * 84 + dy + oy
                db = c * _KH * _KW + dy * _KW
                for h in range(2):
                    v = rows_v[r, pl.ds(db + h * 16, 16)]
                    cur = acc[row, pl.ds(ox + h * 16, 16)]
                    acc[row, pl.ds(ox + h * 16, 16)] = cur + v
        return 0

    lax.fori_loop(0, nrows, _row, 0)
    pltpu.sync_copy(acc, part_ref.at[wid])


def _fold_sc(values, idx):
    k = pl.kernel(
        _fold_sc_body,
        out_type=jax.ShapeDtypeStruct((_NW, _AR, _AC), jnp.float32),
        mesh=plsc.VectorSubcoreMesh(core_axis_name="c", subcore_axis_name="s"),
        scratch_types=[
            pltpu.VMEM((_RPT,), jnp.int32),
            pltpu.VMEM((_GCH, _D), jnp.float32),
            pltpu.VMEM((_AR, _AC), jnp.float32),
            pltpu.SemaphoreType.DMA,
        ],
    )
    return k(values, idx)


def _merge_body(p_ref, out_ref):
    s = jnp.sum(p_ref[...], axis=0)
    folded = jnp.stack(
        [s[c * 84 + _P:c * 84 + _P + _H, _P:_P + _W] for c in range(_C)], 0)
    out_ref[...] = folded / jnp.max(folded)


def _merge(partials):
    return pl.pallas_call(
        _merge_body,
        out_shape=jax.ShapeDtypeStruct((_C, _H, _W), jnp.float32),
    )(partials)


def _fold_body(pat_ref, out_ref, acc):
    i = pl.program_id(0)

    @pl.when(i == 0)
    def _init():
        acc[...] = jnp.zeros((_C, _H + 2 * _P, _W + 2 * _P), jnp.float32)

    blk = pat_ref[...]
    for dx in range(_KW):
        cur = acc[:, pl.ds(i, _OH), pl.ds(dx, _OW)]
        acc[:, pl.ds(i, _OH), pl.ds(dx, _OW)] = cur + blk[:, 0, dx]

    @pl.when(i == _KH - 1)
    def _fin():
        folded = acc[:, _P:_P + _H, _P:_P + _W]
        out_ref[...] = folded / jnp.max(folded)


def _fold(pat5):
    return pl.pallas_call(
        _fold_body,
        grid=(_KH,),
        in_specs=[pl.BlockSpec((_C, 1, _KW, _OH, _OW),
                               lambda i: (0, i, 0, 0, 0))],
        out_specs=pl.BlockSpec((_C, _H, _W), lambda i: (0, 0, 0)),
        out_shape=jax.ShapeDtypeStruct((_C, _H, _W), jnp.float32),
        scratch_shapes=[pltpu.VMEM((_C, _H + 2 * _P, _W + 2 * _P), jnp.float32)],
    )(pat5)


def kernel(image, mem_keys, mem_values):
    img = jnp.transpose(image, (2, 0, 1))
    padded = jnp.pad(img, ((0, 0), (_P, _P), (_P, _P)))
    q = _unfold_sc(padded.reshape(_C * 84 * 84))

    qsq = jnp.sum(q * q, axis=1, keepdims=True)
    ksq = jnp.sum(mem_keys * mem_keys, axis=1)[None, :]
    ksq = jnp.pad(ksq, ((0, 0), (0, _NM * _BM - _M)))

    nn = _nn_search(q.astype(jnp.bfloat16), mem_keys, qsq, ksq)[:, 0]

    partials = _fold_sc(mem_values, nn)
    out = _merge(partials)
    return jnp.transpose(out, (1, 2, 0))


# speedup vs baseline: 103.6135x; 1.0291x over previous
import functools

import jax
import jax.numpy as jnp
from jax import lax
from jax.experimental import pallas as pl
from jax.experimental.pallas import tpu as pltpu
from jax.experimental.pallas import tpu_sc as plsc

_H, _W, _C = 64, 64, 3
_KH, _KW = 32, 32
_P = 10
_OH = _H + 2 * _P - _KH + 1
_OW = _W + 2 * _P - _KW + 1
_Q = _OH * _OW
_D = _C * _KH * _KW
_M = 10000

_QP = 2816
_BQ = 2816
_NQ = _QP // _BQ
_BM = 512
_NM = (_M + _BM - 1) // _BM


def _nn_body(q_ref, k_ref, qsq_ref, ksq_ref, nn_ref, bv, bi):
    mb = pl.program_id(1)

    @pl.when(mb == 0)
    def _init():
        bv[...] = jnp.full((_BQ, 1), jnp.inf, jnp.float32)
        bi[...] = jnp.zeros((_BQ, 1), jnp.int32)

    qk = lax.dot_general(q_ref[...],
                         k_ref[...].astype(jnp.bfloat16),
                         (((1,), (1,)), ((), ())),
                         preferred_element_type=jnp.float32)
    d = (qsq_ref[...] - 2.0 * qk) + ksq_ref[...]
    ids = lax.broadcasted_iota(jnp.int32, d.shape, 1) + mb * _BM
    d = jnp.where(ids < _M, d, jnp.inf)
    dmin = jnp.min(d, axis=1, keepdims=True)
    imin = jnp.min(jnp.where(d == dmin, ids, _NM * _BM), axis=1, keepdims=True)
    take = dmin < bv[...]
    bi[...] = jnp.where(take, imin, bi[...])
    bv[...] = jnp.where(take, dmin, bv[...])

    @pl.when(mb == _NM - 1)
    def _emit():
        nn_ref[...] = bi[...]


def _nn_search(q, keys, qsq, ksq):
    return pl.pallas_call(
        _nn_body,
        grid=(_NQ, _NM),
        in_specs=[
            pl.BlockSpec((_BQ, _D), lambda iq, im: (iq, 0)),
            pl.BlockSpec((_BM, _D), lambda iq, im: (im, 0)),
            pl.BlockSpec((_BQ, 1), lambda iq, im: (iq, 0)),
            pl.BlockSpec((1, _BM), lambda iq, im: (0, im)),
        ],
        out_specs=pl.BlockSpec((_BQ, 1), lambda iq, im: (iq, 0)),
        out_shape=jax.ShapeDtypeStruct((_QP, 1), jnp.int32),
        scratch_shapes=[
            pltpu.VMEM((_BQ, 1), jnp.float32),
            pltpu.VMEM((_BQ, 1), jnp.int32),
        ],
        compiler_params=pltpu.CompilerParams(
            dimension_semantics=("arbitrary", "arbitrary"),
            vmem_limit_bytes=100 * 1024 * 1024),
    )(q, keys, qsq, ksq)


_NW = 32
_BPW = _QP // _NW
_GC = 8
_NCH = _BPW // _GC


def _gather_body(vals_ref, idx_ref, out_ref, idx_v, rows_a, rows_b, sem_a, sem_b):
    wid = lax.axis_index("s") * 2 + lax.axis_index("c")
    base = wid * _BPW
    pltpu.sync_copy(idx_ref.at[pl.ds(base, _BPW)], idx_v)
    bufs = (rows_a, rows_b)
    sems = (sem_a, sem_b)

    def _start(ch):
        return pltpu.async_copy(
            vals_ref.at[idx_v.at[pl.ds(ch * _GC, _GC)]],
            bufs[ch % 2], sems[ch % 2])

    handles = [None] * _NCH
    handles[0] = _start(0)
    for ch in range(_NCH):
        if ch + 1 < _NCH:
            handles[ch + 1] = _start(ch + 1)
        handles[ch].wait()
        pltpu.sync_copy(bufs[ch % 2], out_ref.at[pl.ds(base + ch * _GC, _GC)])


def _sc_gather(values, idx):
    k = pl.kernel(
        _gather_body,
        out_type=jax.ShapeDtypeStruct((_QP, _D), jnp.float32),
        mesh=plsc.VectorSubcoreMesh(core_axis_name="c", subcore_axis_name="s"),
        scratch_types=[
            pltpu.VMEM((_BPW,), jnp.int32),
            pltpu.VMEM((_GC, _D), jnp.float32),
            pltpu.VMEM((_GC, _D), jnp.float32),
            pltpu.SemaphoreType.DMA,
            pltpu.SemaphoreType.DMA,
        ],
    )
    return k(values, idx)


def _unfold_sc_body(img_ref, q_ref, img_v, row_v, sem):
    wid = lax.axis_index("s") * 2 + lax.axis_index("c")
    base = wid * _RPT
    pltpu.sync_copy(img_ref, img_v)

    nrows = _RPT - (_QP - _Q) * (wid // (_NW - 1))

    def _row(i, _):
        r = i - (i // _GCH) * _GCH
        qi = base + i
        oy = lax.shift_right_logical(qi * 19785, 20)
        ox = qi - oy * _OW
        off0 = oy * 84 + ox
        for c in range(_C):
            for dy in range(_KH):
                src = off0 + c * 7056 + dy * 84
                db = c * _KH * _KW + dy * _KW
                for h in range(2):
                    row_v[r, pl.ds(db + h * 16, 16)] = img_v[pl.ds(src + h * 16, 16)]

        @pl.when(i - (i // _GCH) * _GCH == _GCH - 1)
        def _flush():
            pltpu.sync_copy(
                row_v, q_ref.at[pl.ds(pl.multiple_of((i // _GCH) * _GCH + base, 8),
                                      _GCH)])
        return 0

    lax.fori_loop(0, nrows, _row, 0)

    @pl.when(wid == _NW - 1)
    def _tail():
        pltpu.sync_copy(row_v, q_ref.at[pl.ds(pl.multiple_of(base + 80, 8), _GCH)])


def _unfold_sc(padimg):
    k = pl.kernel(
        _unfold_sc_body,
        out_type=jax.ShapeDtypeStruct((_QP, _D), jnp.float32),
        mesh=plsc.VectorSubcoreMesh(core_axis_name="c", subcore_axis_name="s"),
        scratch_types=[
            pltpu.VMEM((_C * 84 * 84,), jnp.float32),
            pltpu.VMEM((_GCH, _D), jnp.float32),
            pltpu.SemaphoreType.DMA,
        ],
    )
    return k(padimg)


_RPT = _QP // _NW
_GCH = 8
_NG = _D // 16
_AR, _AC = _C * 84, 96


def _fold_sc_body(vals_ref, idx_ref, part_ref, idx_v, rows_a, rows_b,
                  acc, sem_a, sem_b):
    wid = lax.axis_index("s") * 2 + lax.axis_index("c")
    base = wid * _RPT
    pltpu.sync_copy(idx_ref.at[pl.ds(base, _RPT)], idx_v)

    def _zero(r, _):
        for g in range(_AC // 16):
            acc[r, pl.ds(g * 16, 16)] = jnp.zeros((16,), jnp.float32)
        return 0

    lax.fori_loop(0, _AR, _zero, 0)

    nrows = _RPT - (_QP - _Q) * (wid // (_NW - 1))

    def _start(ck, buf, sem):
        return pltpu.async_copy(
            vals_ref.at[idx_v.at[pl.ds(pl.multiple_of(ck * _GCH, 8), _GCH)]],
            buf, sem)

    def _accum_row(i, buf):
        r = i - (i // _GCH) * _GCH
        qi = base + i
        oy = lax.shift_right_logical(qi * 19785, 20)
        ox = qi - oy * _OW
        for c in range(_C):
            for dy in range(_KH):
                row = c * 84 + dy + oy
                db = c * _KH * _KW + dy * _KW
                for h in range(2):
                    v = buf[r, pl.ds(db + h * 16, 16)]
                    cur = acc[row, pl.ds(ox + h * 16, 16)]
                    acc[row, pl.ds(ox + h * 16, 16)] = cur + v

    _start(0, rows_a, sem_a)

    def _pair(j, _):
        ca = 2 * j
        _start(ca + 1, rows_b, sem_b)
        pltpu.make_async_copy(
            vals_ref.at[idx_v.at[pl.ds(pl.multiple_of(ca * _GCH, 8), _GCH)]],
            rows_a, sem_a).wait()
        lax.fori_loop(ca * _GCH, (ca + 1) * _GCH,
                      lambda i, c: (_accum_row(i, rows_a), 0)[1], 0)
        _start(ca + 2, rows_a, sem_a)
        pltpu.make_async_copy(
            vals_ref.at[idx_v.at[pl.ds(pl.multiple_of((ca + 1) * _GCH, 8), _GCH)]],
            rows_b, sem_b).wait()
        lax.fori_loop((ca + 1) * _GCH, (ca + 2) * _GCH,
                      lambda i, c: (_accum_row(i, rows_b), 0)[1], 0)
        return 0

    lax.fori_loop(0, (_RPT // _GCH) // 2, _pair, 0)

    pltpu.make_async_copy(
        vals_ref.at[idx_v.at[pl.ds(pl.multiple_of(10 * _GCH, 8), _GCH)]],
        rows_a, sem_a).wait()
    lax.fori_loop(10 * _GCH, nrows,
                  lambda i, c: (_accum_row(i, rows_a), 0)[1], 0)

    pltpu.sync_copy(acc, part_ref.at[wid])


def _fold_sc(values, idx):
    k = pl.kernel(
        _fold_sc_body,
        out_type=jax.ShapeDtypeStruct((_NW, _AR, _AC), jnp.float32),
        mesh=plsc.VectorSubcoreMesh(core_axis_name="c", subcore_axis_name="s"),
        scratch_types=[
            pltpu.VMEM((_RPT,), jnp.int32),
            pltpu.VMEM((_GCH, _D), jnp.float32),
            pltpu.VMEM((_GCH, _D), jnp.float32),
            pltpu.VMEM((_AR, _AC), jnp.float32),
            pltpu.SemaphoreType.DMA,
            pltpu.SemaphoreType.DMA,
        ],
    )
    return k(values, idx)


def _merge_body(p_ref, out_ref):
    s = jnp.sum(p_ref[...], axis=0)
    folded = jnp.stack(
        [s[c * 84 + _P:c * 84 + _P + _H, _P:_P + _W] for c in range(_C)], 0)
    out_ref[...] = folded / jnp.max(folded)


def _merge(partials):
    return pl.pallas_call(
        _merge_body,
        out_shape=jax.ShapeDtypeStruct((_C, _H, _W), jnp.float32),
    )(partials)


def _fold_body(pat_ref, out_ref, acc):
    i = pl.program_id(0)

    @pl.when(i == 0)
    def _init():
        acc[...] = jnp.zeros((_C, _H + 2 * _P, _W + 2 * _P), jnp.float32)

    blk = pat_ref[...]
    for dx in range(_KW):
        cur = acc[:, pl.ds(i, _OH), pl.ds(dx, _OW)]
        acc[:, pl.ds(i, _OH), pl.ds(dx, _OW)] = cur + blk[:, 0, dx]

    @pl.when(i == _KH - 1)
    def _fin():
        folded = acc[:, _P:_P + _H, _P:_P + _W]
        out_ref[...] = folded / jnp.max(folded)


def _fold(pat5):
    return pl.pallas_call(
        _fold_body,
        grid=(_KH,),
        in_specs=[pl.BlockSpec((_C, 1, _KW, _OH, _OW),
                               lambda i: (0, i, 0, 0, 0))],
        out_specs=pl.BlockSpec((_C, _H, _W), lambda i: (0, 0, 0)),
        out_shape=jax.ShapeDtypeStruct((_C, _H, _W), jnp.float32),
        scratch_shapes=[pltpu.VMEM((_C, _H + 2 * _P, _W + 2 * _P), jnp.float32)],
    )(pat5)


def kernel(image, mem_keys, mem_values):
    img = jnp.transpose(image, (2, 0, 1))
    padded = jnp.pad(img, ((0, 0), (_P, _P), (_P, _P)))
    q = _unfold_sc(padded.reshape(_C * 84 * 84))

    qsq = jnp.sum(q * q, axis=1, keepdims=True)
    ksq = jnp.sum(mem_keys * mem_keys, axis=1)[None, :]
    ksq = jnp.pad(ksq, ((0, 0), (0, _NM * _BM - _M)))

    nn = _nn_search(q.astype(jnp.bfloat16), mem_keys, qsq, ksq)[:, 0]

    partials = _fold_sc(mem_values, nn)
    out = _merge(partials)
    return jnp.transpose(out, (1, 2, 0))


# speedup vs baseline: 103.7372x; 1.0012x over previous
import jax
import jax.numpy as jnp
from jax import lax
from jax.experimental import pallas as pl
from jax.experimental.pallas import tpu as pltpu
from jax.experimental.pallas import tpu_sc as plsc

_H, _W, _C = 64, 64, 3
_KH, _KW = 32, 32
_P = 10
_OH = _H + 2 * _P - _KH + 1
_OW = _W + 2 * _P - _KW + 1
_Q = _OH * _OW
_D = _C * _KH * _KW
_M = 10000

_QP = 2816
_BQ = 2816
_NQ = _QP // _BQ
_BM = 512
_NM = (_M + _BM - 1) // _BM


def _nn_body(q_ref, k_ref, qsq_ref, ksq_ref, nn_ref, bv, bi):
    mb = pl.program_id(1)

    @pl.when(mb == 0)
    def _init():
        bv[...] = jnp.full((_BQ, 1), jnp.inf, jnp.float32)
        bi[...] = jnp.zeros((_BQ, 1), jnp.int32)

    qk = lax.dot_general(q_ref[...],
                         k_ref[...].astype(jnp.bfloat16),
                         (((1,), (1,)), ((), ())),
                         preferred_element_type=jnp.float32)
    d = (qsq_ref[...] - 2.0 * qk) + ksq_ref[...]
    ids = lax.broadcasted_iota(jnp.int32, d.shape, 1) + mb * _BM
    d = jnp.where(ids < _M, d, jnp.inf)
    dmin = jnp.min(d, axis=1, keepdims=True)
    imin = jnp.min(jnp.where(d == dmin, ids, _NM * _BM), axis=1, keepdims=True)
    take = dmin < bv[...]
    bi[...] = jnp.where(take, imin, bi[...])
    bv[...] = jnp.where(take, dmin, bv[...])

    @pl.when(mb == _NM - 1)
    def _emit():
        nn_ref[...] = bi[...]


def _nn_search(q, keys, qsq, ksq):
    return pl.pallas_call(
        _nn_body,
        grid=(_NQ, _NM),
        in_specs=[
            pl.BlockSpec((_BQ, _D), lambda iq, im: (iq, 0)),
            pl.BlockSpec((_BM, _D), lambda iq, im: (im, 0)),
            pl.BlockSpec((_BQ, 1), lambda iq, im: (iq, 0)),
            pl.BlockSpec((1, _BM), lambda iq, im: (0, im)),
        ],
        out_specs=pl.BlockSpec((_BQ, 1), lambda iq, im: (iq, 0)),
        out_shape=jax.ShapeDtypeStruct((_QP, 1), jnp.int32),
        scratch_shapes=[
            pltpu.VMEM((_BQ, 1), jnp.float32),
            pltpu.VMEM((_BQ, 1), jnp.int32),
        ],
        compiler_params=pltpu.CompilerParams(
            dimension_semantics=("arbitrary", "arbitrary"),
            vmem_limit_bytes=100 * 1024 * 1024),
    )(q, keys, qsq, ksq)


_NW = 32


def _unfold_sc_body(img_ref, q_ref, img_v, row_v, sem):
    wid = lax.axis_index("s") * 2 + lax.axis_index("c")
    base = wid * _RPT
    pltpu.sync_copy(img_ref, img_v)

    nrows = _RPT - (_QP - _Q) * (wid // (_NW - 1))

    def _row(i, _):
        r = i - (i // _GCH) * _GCH
        qi = base + i
        oy = lax.shift_right_logical(qi * 19785, 20)
        ox = qi - oy * _OW
        off0 = oy * 84 + ox
        for c in range(_C):
            for dy in range(_KH):
                src = off0 + c * 7056 + dy * 84
                db = c * _KH * _KW + dy * _KW
                for h in range(2):
                    row_v[r, pl.ds(db + h * 16, 16)] = img_v[pl.ds(src + h * 16, 16)]

        @pl.when(i - (i // _GCH) * _GCH == _GCH - 1)
        def _flush():
            pltpu.sync_copy(
                row_v, q_ref.at[pl.ds(pl.multiple_of((i // _GCH) * _GCH + base, 8),
                                      _GCH)])
        return 0

    lax.fori_loop(0, nrows, _row, 0)

    @pl.when(wid == _NW - 1)
    def _tail():
        pltpu.sync_copy(row_v, q_ref.at[pl.ds(pl.multiple_of(base + 80, 8), _GCH)])


def _unfold_sc(padimg):
    k = pl.kernel(
        _unfold_sc_body,
        out_type=jax.ShapeDtypeStruct((_QP, _D), jnp.float32),
        mesh=plsc.VectorSubcoreMesh(core_axis_name="c", subcore_axis_name="s"),
        scratch_types=[
            pltpu.VMEM((_C * 84 * 84,), jnp.float32),
            pltpu.VMEM((_GCH, _D), jnp.float32),
            pltpu.SemaphoreType.DMA,
        ],
    )
    return k(padimg)


_RPT = _QP // _NW
_GCH = 8
_NG = _D // 16
_AR, _AC = _C * 84, 96


def _fold_sc_body(vals_ref, idx_ref, part_ref, idx_v, rows_a, rows_b,
                  acc, sem_a, sem_b):
    wid = lax.axis_index("s") * 2 + lax.axis_index("c")
    base = wid * _RPT
    pltpu.sync_copy(idx_ref.at[pl.ds(base, _RPT)], idx_v)

    def _zero(r, _):
        for g in range(_AC // 16):
            acc[r, pl.ds(g * 16, 16)] = jnp.zeros((16,), jnp.float32)
        return 0

    lax.fori_loop(0, _AR, _zero, 0)

    nrows = _RPT - (_QP - _Q) * (wid // (_NW - 1))

    def _start(ck, buf, sem):
        return pltpu.async_copy(
            vals_ref.at[idx_v.at[pl.ds(pl.multiple_of(ck * _GCH, 8), _GCH)]],
            buf, sem)

    def _accum_row(i, buf):
        r = i - (i // _GCH) * _GCH
        qi = base + i
        oy = lax.shift_right_logical(qi * 19785, 20)
        ox = qi - oy * _OW
        for c in range(_C):
            for dy in range(_KH):
                row = c * 84 + dy + oy
                db = c * _KH * _KW + dy * _KW
                for h in range(2):
                    v = buf[r, pl.ds(db + h * 16, 16)]
                    cur = acc[row, pl.ds(ox + h * 16, 16)]
                    acc[row, pl.ds(ox + h * 16, 16)] = cur + v

    _start(0, rows_a, sem_a)

    def _pair(j, _):
        ca = 2 * j
        _start(ca + 1, rows_b, sem_b)
        pltpu.make_async_copy(
            vals_ref.at[idx_v.at[pl.ds(pl.multiple_of(ca * _GCH, 8), _GCH)]],
            rows_a, sem_a).wait()
        lax.fori_loop(ca * _GCH, (ca + 1) * _GCH,
                      lambda i, c: (_accum_row(i, rows_a), 0)[1], 0)
        _start(ca + 2, rows_a, sem_a)
        pltpu.make_async_copy(
            vals_ref.at[idx_v.at[pl.ds(pl.multiple_of((ca + 1) * _GCH, 8), _GCH)]],
            rows_b, sem_b).wait()
        lax.fori_loop((ca + 1) * _GCH, (ca + 2) * _GCH,
                      lambda i, c: (_accum_row(i, rows_b), 0)[1], 0)
        return 0

    lax.fori_loop(0, (_RPT // _GCH) // 2, _pair, 0)

    pltpu.make_async_copy(
        vals_ref.at[idx_v.at[pl.ds(pl.multiple_of(10 * _GCH, 8), _GCH)]],
        rows_a, sem_a).wait()
    lax.fori_loop(10 * _GCH, nrows,
                  lambda i, c: (_accum_row(i, rows_a), 0)[1], 0)

    pltpu.sync_copy(acc, part_ref.at[wid])


def _fold_sc(values, idx):
    k = pl.kernel(
        _fold_sc_body,
        out_type=jax.ShapeDtypeStruct((_NW, _AR, _AC), jnp.float32),
        mesh=plsc.VectorSubcoreMesh(core_axis_name="c", subcore_axis_name="s"),
        scratch_types=[
            pltpu.VMEM((_RPT,), jnp.int32),
            pltpu.VMEM((_GCH, _D), jnp.float32),
            pltpu.VMEM((_GCH, _D), jnp.float32),
            pltpu.VMEM((_AR, _AC), jnp.float32),
            pltpu.SemaphoreType.DMA,
            pltpu.SemaphoreType.DMA,
        ],
    )
    return k(values, idx)


def _merge_body(p_ref, out_ref):
    s = jnp.sum(p_ref[...], axis=0)
    folded = jnp.stack(
        [s[c * 84 + _P:c * 84 + _P + _H, _P:_P + _W] for c in range(_C)], 0)
    out_ref[...] = folded / jnp.max(folded)


def _merge(partials):
    return pl.pallas_call(
        _merge_body,
        out_shape=jax.ShapeDtypeStruct((_C, _H, _W), jnp.float32),
    )(partials)


def kernel(image, mem_keys, mem_values):
    img = jnp.transpose(image, (2, 0, 1))
    padded = jnp.pad(img, ((0, 0), (_P, _P), (_P, _P)))
    q = _unfold_sc(padded.reshape(_C * 84 * 84))

    qsq = jnp.sum(q * q, axis=1, keepdims=True)
    ksq = jnp.sum(mem_keys * mem_keys, axis=1)[None, :]
    ksq = jnp.pad(ksq, ((0, 0), (0, _NM * _BM - _M)))

    nn = _nn_search(q.astype(jnp.bfloat16), mem_keys, qsq, ksq)[:, 0]

    partials = _fold_sc(mem_values, nn)
    out = _merge(partials)
    return jnp.transpose(out, (1, 2, 0))
